# baseline (speedup 1.0000x reference)
import jax
import jax.numpy as jnp
from jax import lax
from jax.experimental import pallas as pl
from jax.experimental.pallas import tpu as pltpu
from jax.experimental.pallas import tpu_sc as plsc

_H = 16
_TBL = 128
_CELL0 = 508


def _sc_body(tsq_v, tsk_v, ptab_v, wt_v, out_v):
    ncols = tsk_v.shape[1]
    tq = tsq_v[0, pl.ds(0, 16)]

    @pl.loop(0, ncols, step=16)
    def _(c):
        tk = tsk_v[0, pl.ds(c, 16)]
        td = jnp.maximum(jnp.abs(tk - tq), 1)
        tdf = td.astype(jnp.float32)
        bits = lax.bitcast_convert_type(tdf, jnp.int32)
        cell = (bits >> 21) - _CELL0
        pk = plsc.load_gather(ptab_v, [cell])
        base = pk & 127
        thr = pk >> 7
        bk = base + (thr < td).astype(jnp.int32)
        for p in range(_H // 2):
            word = plsc.load_gather(wt_v, [bk + p * _TBL])
            lo = lax.bitcast_convert_type(word << 16, jnp.float32)
            hi = lax.bitcast_convert_type(word & jnp.int32(-65536), jnp.float32)
            out_v[2 * p, 0, pl.ds(c, 16)] = lo
            out_v[2 * p + 1, 0, pl.ds(c, 16)] = hi


def kernel(L, timestamps, time_bias_weight, time_boundaries):
    del L
    B, L = timestamps.shape
    tsq = jnp.broadcast_to(timestamps.reshape(L, 1), (L, 16))
    tsk = timestamps.reshape(1, L)

    cell_ids = jnp.arange(_TBL, dtype=jnp.uint32)
    cell_lo = lax.bitcast_convert_type((cell_ids + _CELL0) << 21, jnp.float32)
    cell_hi = lax.bitcast_convert_type((cell_ids + _CELL0 + 1) << 21, jnp.float32)
    base = jnp.searchsorted(time_boundaries, cell_lo, side="left").astype(jnp.int32)
    b_pad = jnp.concatenate([time_boundaries, jnp.full((1,), 1e30, jnp.float32)])
    cand = b_pad[base]
    thr = jnp.where(cand < cell_hi, cand, jnp.float32(2**23)).astype(jnp.int32)
    ptab = ((thr << 7) | base).reshape(_TBL)

    wb = time_bias_weight.astype(jnp.bfloat16)
    wu = lax.bitcast_convert_type(wb, jnp.uint16).astype(jnp.uint32)
    wpair = (wu[:, 1::2] << 16) | wu[:, 0::2]
    wt = (
        jnp.zeros((_H // 2, _TBL), jnp.uint32)
        .at[:, : time_bias_weight.shape[0]]
        .set(wpair.T)
        .astype(jnp.int32)
        .reshape(_H // 2 * _TBL)
    )

    mesh = plsc.VectorSubcoreMesh(core_axis_name="c", subcore_axis_name="s")

    @pl.kernel(
        out_type=jax.ShapeDtypeStruct((_H, L, L), jnp.float32),
        mesh=mesh,
        compiler_params=pltpu.CompilerParams(needs_layout_passes=False),
    )
    def sc_kernel(tsq_hbm, tsk_hbm, ptab_hbm, wt_hbm, out_hbm):
        pltpu.emit_pipeline(
            _sc_body,
            grid=(L,),
            in_specs=[
                pl.BlockSpec((1, 16), lambda i: (i, 0)),
                pl.BlockSpec((1, L), lambda i: (0, 0)),
                pl.BlockSpec((_TBL,), lambda i: (0,)),
                pl.BlockSpec((_H // 2 * _TBL,), lambda i: (0,)),
            ],
            out_specs=[pl.BlockSpec((_H, 1, L), lambda i: (0, i, 0))],
            core_axis_name=("c", "s"),
            dimension_semantics=(pltpu.PARALLEL,),
        )(tsq_hbm, tsk_hbm, ptab_hbm, wt_hbm, out_hbm)

    out = sc_kernel(tsq, tsk, ptab, wt)
    return out.reshape(B, _H, L, L)


# speedup vs baseline: 3.7737x; 3.7737x over previous
import jax
import jax.numpy as jnp
from jax import lax
from jax.experimental import pallas as pl
from jax.experimental.pallas import tpu as pltpu

_H = 16
_TI = 64
_TBL = 128
_CELL0 = 508


def _bias_kernel(cell_ref, tsq_ref, tsk_ref, wt_ref, out_ref):
    rows = tsq_ref.shape[0]
    tq = tsq_ref[...]
    tk = tsk_ref[...]
    td = jnp.maximum(jnp.abs(tq - tk), 1)
    tdf = td.astype(jnp.float32)
    bits = lax.bitcast_convert_type(tdf, jnp.int32)
    cell = (bits >> 21) - _CELL0
    ctab = jnp.broadcast_to(cell_ref[...], (rows, _TBL))
    packed = jnp.take_along_axis(ctab, cell, axis=1)
    base = packed & 127
    thr = packed >> 7
    bk = base + (thr < td).astype(jnp.int32)
    npair = _H // 2
    tab3 = jnp.broadcast_to(wt_ref[...][:, None, :], (npair, rows, _TBL))
    idx3 = jnp.broadcast_to(bk[None], (npair,) + bk.shape)
    words = jnp.take_along_axis(tab3, idx3, axis=2)
    for p in range(npair):
        word = words[p]
        out_ref[0, 2 * p] = lax.bitcast_convert_type(word << 16, jnp.float32)
        out_ref[0, 2 * p + 1] = lax.bitcast_convert_type(
            word & jnp.int32(-65536), jnp.float32
        )


def kernel(L, timestamps, time_bias_weight, time_boundaries):
    del L
    B, L = timestamps.shape
    nb = time_boundaries.shape[0]
    tsq = timestamps.reshape(L, 1)
    tsk = timestamps.reshape(1, L)

    cell_ids = jnp.arange(_TBL, dtype=jnp.uint32)
    cell_lo = lax.bitcast_convert_type((cell_ids + _CELL0) << 21, jnp.float32)
    cell_hi = lax.bitcast_convert_type((cell_ids + _CELL0 + 1) << 21, jnp.float32)
    base = jnp.searchsorted(time_boundaries, cell_lo, side="left").astype(jnp.int32)
    b_pad = jnp.concatenate([time_boundaries, jnp.full((1,), 1e30, jnp.float32)])
    cand = b_pad[base]
    thr = jnp.where(cand < cell_hi, cand, jnp.float32(2**23)).astype(jnp.int32)
    packed = ((thr << 7) | base).reshape(1, _TBL)

    wb = time_bias_weight.astype(jnp.bfloat16)
    wu = lax.bitcast_convert_type(wb, jnp.uint16).astype(jnp.uint32)
    wpair = (wu[:, 1::2] << 16) | wu[:, 0::2]
    wt = (
        jnp.zeros((_H // 2, _TBL), jnp.uint32)
        .at[:, : time_bias_weight.shape[0]]
        .set(wpair.T)
        .astype(jnp.int32)
    )

    grid = (L // _TI,)
    out = pl.pallas_call(
        _bias_kernel,
        grid=grid,
        in_specs=[
            pl.BlockSpec((1, _TBL), lambda i: (0, 0)),
            pl.BlockSpec((_TI, 1), lambda i: (i, 0)),
            pl.BlockSpec((1, L), lambda i: (0, 0)),
            pl.BlockSpec((_H // 2, _TBL), lambda i: (0, 0)),
        ],
        out_specs=pl.BlockSpec((1, _H, _TI, L), lambda i: (0, 0, i, 0)),
        out_shape=jax.ShapeDtypeStruct((B, _H, L, L), jnp.float32),
    )(packed, tsq, tsk, wt)
    return out
